# baseline (speedup 1.0000x reference)
import functools

import jax
import jax.numpy as jnp
from jax import lax
from jax.experimental import pallas as pl
from jax.experimental.pallas import tpu as pltpu
from jax.experimental.pallas import tpu_sc as plsc

_INFO = plsc.get_sparse_core_info()
_NC = _INFO.num_cores
_NS = _INFO.num_subcores
_NW = _NC * _NS
_LANES = _INFO.num_lanes

_CHUNK = 320
_NBUF = 2


def _norm_table_body(tab_ref, gamma_ref, beta_ref, out_ref):
    x = tab_ref[...]
    mean = jnp.mean(x, axis=-1, keepdims=True)
    var = jnp.mean((x - mean) * (x - mean), axis=-1, keepdims=True)
    inv = lax.rsqrt(var + 1e-12)
    out_ref[...] = (x - mean) * inv * gamma_ref[...] + beta_ref[...]


def _norm_table(snp_table, ln_gamma, ln_beta):
    v, d = snp_table.shape
    return pl.pallas_call(
        _norm_table_body,
        out_shape=jax.ShapeDtypeStruct((v, d), jnp.float32),
    )(snp_table, ln_gamma.reshape(1, d), ln_beta.reshape(1, d))


def _make_expand(n_tokens, n_rows, d):
    assert n_tokens % (_NW * _CHUNK) == 0
    per_w = n_tokens // _NW
    n_chunks = per_w // _CHUNK
    n_grp = _CHUNK // _LANES
    assert n_chunks % _NBUF == 0
    mesh = plsc.VectorSubcoreMesh(core_axis_name="c", subcore_axis_name="s")

    @functools.partial(
        pl.kernel,
        out_type=jax.ShapeDtypeStruct((n_tokens, d), jnp.float32),
        mesh=mesh,
        compiler_params=pltpu.CompilerParams(needs_layout_passes=False),
        scratch_types=[
            pltpu.VMEM((per_w,), jnp.int32),
            pltpu.VMEM((n_rows, d), jnp.float32),
            pltpu.VMEM((d, _LANES), jnp.float32),
            pltpu.VMEM((_NBUF * _CHUNK, d), jnp.float32),
            pltpu.SemaphoreType.DMA,
            pltpu.SemaphoreType.DMA,
        ],
    )
    def expand_kernel(idx_hbm, tab_hbm, out_hbm, idx_v, tab_v, tab_t, rows_v, sem0, sem1):
        wid = lax.axis_index("s") * _NC + lax.axis_index("c")
        pltpu.sync_copy(idx_hbm.at[wid], idx_v)
        pltpu.sync_copy(tab_hbm, tab_v)
        base = wid * per_w
        sems = [sem0, sem1]
        iota = lax.iota(jnp.int32, _LANES)

        for v in range(n_rows):
            for cg in range(d // _LANES):
                y = tab_v[v, pl.ds(cg * _LANES, _LANES)]
                cvec = iota + cg * _LANES
                vvec = jnp.full((_LANES,), v, jnp.int32)
                plsc.store_scatter(tab_t, [cvec, vvec], y)

        def build(k, buf):
            rowbase = buf * _CHUNK

            def gbody(g, carry):
                off = k * _CHUNK + g * _LANES
                iv = idx_v[pl.ds(off, _LANES)]
                rvec = iota + (rowbase + g * _LANES)
                for c in range(d):
                    cvec = jnp.full((_LANES,), c, jnp.int32)
                    yc = plsc.load_gather(tab_t, [cvec, iv])
                    plsc.store_scatter(rows_v, [rvec, cvec], yc)
                return carry

            lax.fori_loop(0, n_grp, gbody, 0)

        def store(k, buf):
            pltpu.async_copy(
                rows_v.at[pl.ds(buf * _CHUNK, _CHUNK)],
                out_hbm.at[pl.ds(base + k * _CHUNK, _CHUNK)],
                sems[buf],
            )

        def drain(buf):
            pltpu.make_async_copy(
                rows_v.at[pl.ds(buf * _CHUNK, _CHUNK)],
                out_hbm.at[pl.ds(0, _CHUNK)],
                sems[buf],
            ).wait()

        for buf in range(_NBUF):
            build(buf, buf)
            store(buf, buf)

        def outer(k2, carry):
            for buf in range(_NBUF):
                k = k2 * _NBUF + buf
                drain(buf)
                build(k, buf)
                store(k, buf)
            return carry

        lax.fori_loop(1, n_chunks // _NBUF, outer, 0)
        for buf in range(_NBUF):
            drain(buf)

    return expand_kernel


def kernel(snp, is_padding, snp_table, ln_gamma, ln_beta):
    b, l = snp.shape
    v, d = snp_table.shape
    n = b * l
    ntab = _norm_table(snp_table, ln_gamma, ln_beta)
    idx = snp.reshape(_NW, n // _NW).astype(jnp.int32)
    out = _make_expand(n, v, d)(idx, ntab)
    return out.reshape(b, l, d), is_padding


# speedup vs baseline: 7.3257x; 7.3257x over previous
import functools

import jax
import jax.numpy as jnp
from jax import lax
from jax.experimental import pallas as pl
from jax.experimental.pallas import tpu as pltpu
from jax.experimental.pallas import tpu_sc as plsc

_INFO = plsc.get_sparse_core_info()
_NC = _INFO.num_cores
_NS = _INFO.num_subcores
_NW = _NC * _NS
_LANES = _INFO.num_lanes

_CHUNK = 128
_NBUF = 2


def _norm_table_body(tab_ref, gamma_ref, beta_ref, out_ref):
    x = tab_ref[...]
    mean = jnp.mean(x, axis=-1, keepdims=True)
    var = jnp.mean((x - mean) * (x - mean), axis=-1, keepdims=True)
    inv = lax.rsqrt(var + 1e-12)
    out_ref[...] = (x - mean) * inv * gamma_ref[...] + beta_ref[...]


def _norm_table(snp_table, ln_gamma, ln_beta):
    v, d = snp_table.shape
    return pl.pallas_call(
        _norm_table_body,
        out_shape=jax.ShapeDtypeStruct((v, d), jnp.float32),
    )(snp_table, ln_gamma.reshape(1, d), ln_beta.reshape(1, d))


def _make_expand(n_tokens, n_rows, d):
    assert n_tokens % (_NW * _CHUNK) == 0
    per_w = n_tokens // _NW
    n_chunks = per_w // _CHUNK
    mesh = plsc.VectorSubcoreMesh(core_axis_name="c", subcore_axis_name="s")

    @functools.partial(
        pl.kernel,
        out_type=jax.ShapeDtypeStruct((n_tokens, d), jnp.float32),
        mesh=mesh,
        compiler_params=pltpu.CompilerParams(needs_layout_passes=False),
        scratch_types=[
            pltpu.VMEM((n_chunks, _CHUNK), jnp.int32),
            pltpu.VMEM_SHARED((n_rows, d), jnp.float32),
            pltpu.VMEM((_NBUF, _CHUNK, d), jnp.float32),
            pltpu.SemaphoreType.DMA,
            pltpu.SemaphoreType.DMA,
            pltpu.SemaphoreType.DMA,
        ],
    )
    def expand_kernel(idx_hbm, tab_hbm, out_hbm, idx_v, tab_v, rows_v, semg, sem0, sem1):
        wid = lax.axis_index("s") * _NC + lax.axis_index("c")
        pltpu.sync_copy(idx_hbm.at[wid], idx_v)

        @pl.when(lax.axis_index("s") == 0)
        def _():
            pltpu.sync_copy(tab_hbm, tab_v)

        plsc.subcore_barrier()
        base = wid * per_w
        sems = [sem0, sem1]

        def gather(j, buf):
            pltpu.async_copy(tab_v.at[idx_v.at[j]], rows_v.at[buf], semg).wait()

        def store(j, buf):
            pltpu.async_copy(
                rows_v.at[buf],
                out_hbm.at[pl.ds(base + j * _CHUNK, _CHUNK)],
                sems[buf],
            )

        def drain(buf):
            pltpu.make_async_copy(
                rows_v.at[buf],
                out_hbm.at[pl.ds(0, _CHUNK)],
                sems[buf],
            ).wait()

        for buf in range(_NBUF):
            gather(buf, buf)
            store(buf, buf)

        def outer(k2, carry):
            for buf in range(_NBUF):
                j = k2 * _NBUF + buf
                drain(buf)
                gather(j, buf)
                store(j, buf)
            return carry

        lax.fori_loop(1, n_chunks // _NBUF, outer, 0)
        for buf in range(_NBUF):
            drain(buf)

    return expand_kernel


def kernel(snp, is_padding, snp_table, ln_gamma, ln_beta):
    b, l = snp.shape
    v, d = snp_table.shape
    n = b * l
    ntab = _norm_table(snp_table, ln_gamma, ln_beta)
    idx = snp.reshape(_NW, (n // _NW) // _CHUNK, _CHUNK).astype(jnp.int32)
    out = _make_expand(n, v, d)(idx, ntab)
    return out.reshape(b, l, d), is_padding
